# baseline (speedup 1.0000x reference)
import functools

import jax
import jax.numpy as jnp
from jax import lax
from jax.experimental import pallas as pl
from jax.experimental.pallas import tpu as pltpu
from jax.experimental.pallas import tpu_sc as plsc

_NC = 2
_NS = 16
_NW = _NC * _NS

_B = 80
_WR = 80

_F32 = jnp.float32


def _mesh():
    return plsc.VectorSubcoreMesh(core_axis_name="c", subcore_axis_name="s")


def _zero_tmp(tmp_v, rows, lane_chunks):
    def zrow(r, carry):
        for j in range(lane_chunks):
            tmp_v[r, pl.ds(j * 16, 16)] = jnp.zeros((16,), _F32)
        return carry
    lax.fori_loop(0, rows, zrow, 0)


def _deg_body(n_pad, npw, bpw, srcs3, dsts3, out, srcs_v, dsts_v, ones_v,
              zz_v, wb_v, dega, degb):
    cid = lax.axis_index("c")
    sid = lax.axis_index("s")
    wid = cid * _NS + sid
    for j in range(112 // 16):
        ones_v[pl.ds(j * 16, 16)] = jnp.full((16,), 1.0, _F32)
    for j in range(npw // 16):
        zz_v[pl.ds(j * 16, 16)] = jnp.zeros((16,), _F32)
    pltpu.sync_copy(srcs3.at[wid], srcs_v)
    pltpu.sync_copy(dsts3.at[wid], dsts_v)
    zoff = pl.multiple_of(sid * npw, 128)
    pltpu.sync_copy(zz_v, dega.at[pl.ds(zoff, npw)])
    pltpu.sync_copy(zz_v, degb.at[pl.ds(zoff, npw)])
    plsc.subcore_barrier()

    def ebody(b, carry):
        pltpu.sync_copy(ones_v.at[pl.ds(0, _B)], dega.at[srcs_v.at[b]], add=True)
        pltpu.sync_copy(ones_v.at[pl.ds(0, _B)], degb.at[dsts_v.at[b]], add=True)
        return carry
    lax.fori_loop(0, bpw, ebody, 0)
    plsc.subcore_barrier()
    o_a = pl.multiple_of((cid * 2 + 0) * n_pad + sid * npw, 128)
    o_b = pl.multiple_of((cid * 2 + 1) * n_pad + sid * npw, 128)
    pltpu.sync_copy(dega.at[pl.ds(zoff, npw)], wb_v)
    pltpu.sync_copy(wb_v, out.at[pl.ds(o_a, npw)])
    pltpu.sync_copy(degb.at[pl.ds(zoff, npw)], wb_v)
    pltpu.sync_copy(wb_v, out.at[pl.ds(o_b, npw)])


def _deg_call(srcs3, dsts3, n_pad, bpw):
    npw = n_pad // _NS
    body = functools.partial(_deg_body, n_pad, npw, bpw)
    return pl.kernel(
        body,
        out_type=jax.ShapeDtypeStruct((4 * n_pad,), _F32),
        mesh=_mesh(),
        scratch_types=[
            pltpu.VMEM((bpw, _B), jnp.int32),
            pltpu.VMEM((bpw, _B), jnp.int32),
            pltpu.VMEM((112,), _F32),
            pltpu.VMEM((npw,), _F32),
            pltpu.VMEM((npw,), _F32),
            pltpu.VMEM_SHARED((n_pad,), _F32),
            pltpu.VMEM_SHARED((n_pad,), _F32),
        ],
    )(srcs3, dsts3)


def _spmm_body(n_pad, bpw, n_chunks, tab, pk3, out, pk_v, s0v, s1v, s2v,
               d0v, d1v, d2v, r0, r1, r2, acc, g0, g1, g2, ss0, ss1, ss2):
    svs = (s0v, s1v, s2v)
    dvs = (d0v, d1v, d2v)
    rows = (r0, r1, r2)
    gs = (g0, g1, g2)
    sss = (ss0, ss1, ss2)
    cid = lax.axis_index("c")
    sid = lax.axis_index("s")
    wid = cid * _NS + sid
    rps = n_pad // _NS
    nchk = rps // _WR

    def _unp_src(b, sref):
        for j in range(_B // 16):
            p = pk_v[b, pl.ds(j * 16, 16)]
            sref[pl.ds(j * 16, 16)] = lax.bitwise_and(p, 16383)

    def _unp_dst(b, dref):
        for j in range(_B // 16):
            p = pk_v[b, pl.ds(j * 16, 16)]
            dref[pl.ds(j * 16, 16)] = lax.shift_right_logical(p, 14)

    pltpu.sync_copy(pk3.at[wid], pk_v)
    _zero_tmp(r0, _WR, 8)

    for c in range(n_chunks):
        xn_c = tab.at[c]
        if c > 0:
            _zero_tmp(r0, _WR, 8)
        for i in range(nchk):
            ro = pl.multiple_of(sid * rps + i * _WR, 16)
            pltpu.sync_copy(r0.at[pl.ds(0, _WR), :], acc.at[pl.ds(ro, _WR), :])
        plsc.subcore_barrier()

        def step(b, j, drain, fire):
            jn = (j + 1) % 3
            pltpu.make_async_copy(xn_c.at[svs[j]], rows[j], gs[j]).wait()
            _unp_dst(b, dvs[j])
            pltpu.async_copy(rows[j], acc.at[dvs[j]], sss[j], add=True)
            if fire:
                _unp_src(b + 1, svs[jn])
            if drain:
                pltpu.make_async_copy(rows[jn], acc.at[dvs[jn]],
                                      sss[jn]).wait()
            if fire:
                pltpu.async_copy(xn_c.at[svs[jn]], rows[jn], gs[jn])

        _unp_src(0, s0v)
        pltpu.async_copy(xn_c.at[s0v], r0, g0)
        step(0, 0, drain=False, fire=True)
        step(1, 1, drain=False, fire=True)

        def ebody(t, carry):
            b = 3 * t + 2
            step(b, 2, drain=True, fire=True)
            step(b + 1, 0, drain=True, fire=True)
            step(b + 2, 1, drain=True, fire=True)
            return carry
        lax.fori_loop(0, (bpw - 5) // 3, ebody, 0)
        step(bpw - 3, 2, drain=True, fire=True)
        step(bpw - 2, 0, drain=True, fire=True)
        step(bpw - 1, 1, drain=True, fire=False)
        pltpu.make_async_copy(r0, acc.at[d0v], ss0).wait()
        pltpu.make_async_copy(r1, acc.at[d1v], ss1).wait()

        plsc.subcore_barrier()
        for i in range(nchk):
            ro = pl.multiple_of(sid * rps + i * _WR, 16)
            pltpu.sync_copy(acc.at[pl.ds(ro, _WR), :], r0.at[pl.ds(0, _WR), :])
            pltpu.sync_copy(r0.at[pl.ds(0, _WR), :],
                            out.at[cid, c, pl.ds(ro, _WR), :])


def _spmm_call(tab, pk3, n_pad, bpw):
    n_chunks = tab.shape[0]
    body = functools.partial(_spmm_body, n_pad, bpw, n_chunks)
    return pl.kernel(
        body,
        out_type=jax.ShapeDtypeStruct((_NC, n_chunks, n_pad, 128), _F32),
        mesh=_mesh(),
        scratch_types=(
            [pltpu.VMEM((bpw, _B), jnp.int32)]
            + [pltpu.VMEM((_B,), jnp.int32) for _ in range(6)]
            + [pltpu.VMEM((_B, 128), _F32) for _ in range(3)]
            + [pltpu.VMEM_SHARED((n_pad, 128), _F32)]
            + [pltpu.SemaphoreType.DMA for _ in range(6)]
        ),
    )(tab, pk3)


def _norm_body(deg4_ref, x_ref, xn_ref, io_ref, ii_ref):
    d = deg4_ref[...]
    dout = d[:, 0:1] + d[:, 2:3]
    din = d[:, 1:2] + d[:, 3:4]
    io = lax.rsqrt(jnp.maximum(dout, 1.0))
    ii = lax.rsqrt(jnp.maximum(din, 1.0))
    xn_ref[...] = x_ref[...] * io
    io_ref[...] = io
    ii_ref[...] = ii


def _norm_call(deg4, x, n, rb):
    grid = (n // rb,)
    return pl.pallas_call(
        _norm_body,
        grid=grid,
        in_specs=[
            pl.BlockSpec((rb, 4), lambda i: (i, 0)),
            pl.BlockSpec((rb, 128), lambda i: (i, 0)),
        ],
        out_specs=[
            pl.BlockSpec((rb, 128), lambda i: (i, 0)),
            pl.BlockSpec((rb, 1), lambda i: (i, 0)),
            pl.BlockSpec((rb, 1), lambda i: (i, 0)),
        ],
        out_shape=[
            jax.ShapeDtypeStruct((n, 128), _F32),
            jax.ShapeDtypeStruct((n, 1), _F32),
            jax.ShapeDtypeStruct((n, 1), _F32),
        ],
    )(deg4, x)


def _l1_body(u2_ref, w1_ref, b1_ref, ii_ref, io_ref, zn_ref):
    um = u2_ref[0] + u2_ref[1]
    y = jnp.dot(um, w1_ref[...], preferred_element_type=_F32,
                precision=lax.Precision.HIGHEST)
    y = y * ii_ref[...] + b1_ref[...]
    zn_ref[0] = jnp.maximum(y, 0.0) * io_ref[...]


def _l1_call(u2, w1, b1r, ii, io, n, rb):
    grid = (4, n // rb)
    return pl.pallas_call(
        _l1_body,
        grid=grid,
        in_specs=[
            pl.BlockSpec((2, rb, 128), lambda k, i: (0, i, 0)),
            pl.BlockSpec((128, 128), lambda k, i: (0, k)),
            pl.BlockSpec((1, 128), lambda k, i: (0, k)),
            pl.BlockSpec((rb, 1), lambda k, i: (i, 0)),
            pl.BlockSpec((rb, 1), lambda k, i: (i, 0)),
        ],
        out_specs=pl.BlockSpec((1, rb, 128), lambda k, i: (k, i, 0)),
        out_shape=jax.ShapeDtypeStruct((4, n, 128), _F32),
    )(u2, w1, b1r, ii, io)


def _l2_body(v2_ref, w2_ref, b2_ref, ii_ref, out_ref):
    acc = jnp.zeros(out_ref.shape, out_ref.dtype)
    for k in range(4):
        vk = v2_ref[0, k] + v2_ref[1, k]
        acc = acc + jnp.dot(vk, w2_ref[k], preferred_element_type=_F32,
                            precision=lax.Precision.HIGHEST)
    out_ref[...] = acc * ii_ref[...] + b2_ref[...]


def _l2_call(v2, w2r, b2r, ii, n, rb):
    grid = (n // rb,)
    return pl.pallas_call(
        _l2_body,
        grid=grid,
        in_specs=[
            pl.BlockSpec((2, 4, rb, 128), lambda i: (0, 0, i, 0)),
            pl.BlockSpec((4, 128, 512), lambda i: (0, 0, 0)),
            pl.BlockSpec((1, 512), lambda i: (0, 0)),
            pl.BlockSpec((rb, 1), lambda i: (i, 0)),
        ],
        out_specs=pl.BlockSpec((rb, 512), lambda i: (i, 0)),
        out_shape=jax.ShapeDtypeStruct((n, 512), _F32),
    )(v2, w2r, b2r, ii)


def kernel(features, edge_index, W1, b1, W2, b2):
    n, d_in = features.shape
    e = edge_index.shape[1]
    d_h = W1.shape[1]
    assert d_in == 128 and d_h == 512
    assert e % (_NW * _B) == 0
    bpw = e // (_NW * _B)
    n_pad = ((n + _NS * 128 - 1) // (_NS * 128)) * (_NS * 128)
    rb = 400
    assert n % rb == 0

    src = edge_index[0]
    dst = edge_index[1]
    srcs3 = src.reshape(_NW, bpw, _B)
    dsts3 = dst.reshape(_NW, bpw, _B)
    pk3 = (src + dst * 16384).reshape(_NW, bpw, _B)

    deg = _deg_call(srcs3, dsts3, n_pad, bpw)
    deg4 = jnp.transpose(deg.reshape(4, n_pad)[:, :n])

    xn, io, ii = _norm_call(deg4, features, n, rb)

    u2 = _spmm_call(xn.reshape(1, n, 128), pk3, n_pad, bpw)
    u2 = u2.reshape(_NC, n_pad, 128)

    zn = _l1_call(u2, W1, b1.reshape(1, d_h), ii, io, n, rb)

    v2 = _spmm_call(zn, pk3, n_pad, bpw)

    out = _l2_call(v2, W2.reshape(4, 128, d_h), b2.reshape(1, d_h), ii, n, rb)
    return out


# speedup vs baseline: 1.2568x; 1.2568x over previous
import functools

import jax
import jax.numpy as jnp
from jax import lax
from jax.experimental import pallas as pl
from jax.experimental.pallas import tpu as pltpu
from jax.experimental.pallas import tpu_sc as plsc

_NC = 2
_NS = 16
_NW = _NC * _NS

_B = 80
_WR = 80

_F32 = jnp.float32


def _mesh():
    return plsc.VectorSubcoreMesh(core_axis_name="c", subcore_axis_name="s")


def _zero_tmp(tmp_v, rows, lane_chunks):
    def zrow(r, carry):
        for j in range(lane_chunks):
            tmp_v[r, pl.ds(j * 16, 16)] = jnp.zeros((16,), _F32)
        return carry
    lax.fori_loop(0, rows, zrow, 0)


def _deg_body(n_pad, npw, bpw, srcs3, dsts3, out, srcs_v, dsts_v, ones_v,
              zz_v, wb_v, dega, degb):
    cid = lax.axis_index("c")
    sid = lax.axis_index("s")
    wid = cid * _NS + sid
    for j in range(112 // 16):
        ones_v[pl.ds(j * 16, 16)] = jnp.full((16,), 1.0, _F32)
    for j in range(npw // 16):
        zz_v[pl.ds(j * 16, 16)] = jnp.zeros((16,), _F32)
    pltpu.sync_copy(srcs3.at[wid], srcs_v)
    pltpu.sync_copy(dsts3.at[wid], dsts_v)
    zoff = pl.multiple_of(sid * npw, 128)
    pltpu.sync_copy(zz_v, dega.at[pl.ds(zoff, npw)])
    pltpu.sync_copy(zz_v, degb.at[pl.ds(zoff, npw)])
    plsc.subcore_barrier()

    def ebody(b, carry):
        pltpu.sync_copy(ones_v.at[pl.ds(0, _B)], dega.at[srcs_v.at[b]], add=True)
        pltpu.sync_copy(ones_v.at[pl.ds(0, _B)], degb.at[dsts_v.at[b]], add=True)
        return carry
    lax.fori_loop(0, bpw, ebody, 0)
    plsc.subcore_barrier()
    o_a = pl.multiple_of((cid * 2 + 0) * n_pad + sid * npw, 128)
    o_b = pl.multiple_of((cid * 2 + 1) * n_pad + sid * npw, 128)
    pltpu.sync_copy(dega.at[pl.ds(zoff, npw)], wb_v)
    pltpu.sync_copy(wb_v, out.at[pl.ds(o_a, npw)])
    pltpu.sync_copy(degb.at[pl.ds(zoff, npw)], wb_v)
    pltpu.sync_copy(wb_v, out.at[pl.ds(o_b, npw)])


def _deg_call(srcs3, dsts3, n_pad, bpw):
    npw = n_pad // _NS
    body = functools.partial(_deg_body, n_pad, npw, bpw)
    return pl.kernel(
        body,
        out_type=jax.ShapeDtypeStruct((4 * n_pad,), _F32),
        mesh=_mesh(),
        scratch_types=[
            pltpu.VMEM((bpw, _B), jnp.int32),
            pltpu.VMEM((bpw, _B), jnp.int32),
            pltpu.VMEM((112,), _F32),
            pltpu.VMEM((npw,), _F32),
            pltpu.VMEM((npw,), _F32),
            pltpu.VMEM_SHARED((n_pad,), _F32),
            pltpu.VMEM_SHARED((n_pad,), _F32),
        ],
    )(srcs3, dsts3)


def _spmm_body(n_pad, bpw, n_chunks, tab, pk3, out, pk_v, s0v, s1v, s2v,
               d0v, d1v, d2v, r0, r1, r2, acc, g0, g1, g2, ss0, ss1, ss2):
    svs = (s0v, s1v, s2v)
    dvs = (d0v, d1v, d2v)
    rows = (r0, r1, r2)
    gs = (g0, g1, g2)
    sss = (ss0, ss1, ss2)
    cid = lax.axis_index("c")
    sid = lax.axis_index("s")
    wid = cid * _NS + sid
    rps = n_pad // _NS
    nchk = rps // _WR

    def _unp_src(b, sref):
        for j in range(_B // 16):
            p = pk_v[b, pl.ds(j * 16, 16)]
            sref[pl.ds(j * 16, 16)] = lax.bitwise_and(p, 16383)

    def _unp_dst(b, dref):
        for j in range(_B // 16):
            p = pk_v[b, pl.ds(j * 16, 16)]
            dref[pl.ds(j * 16, 16)] = lax.shift_right_logical(p, 14)

    pltpu.sync_copy(pk3.at[wid], pk_v)
    _zero_tmp(r0, _WR, 8)

    for c in range(n_chunks):
        xn_c = tab.at[c]
        if c > 0:
            _zero_tmp(r0, _WR, 8)
        for i in range(nchk):
            ro = pl.multiple_of(sid * rps + i * _WR, 16)
            pltpu.sync_copy(r0.at[pl.ds(0, _WR), :], acc.at[pl.ds(ro, _WR), :])
        plsc.subcore_barrier()

        _unp_src(0, s0v)
        pltpu.async_copy(xn_c.at[s0v], r0, g0)

        def ebody(t, carry):
            b0 = 2 * t
            _unp_src(b0 + 1, s1v)
            pltpu.async_copy(xn_c.at[s1v], r1, g1)
            _unp_dst(b0, d0v)
            pltpu.make_async_copy(xn_c.at[s0v], r0, g0).wait()
            pltpu.sync_copy(r0, acc.at[d0v], add=True)
            _unp_src(b0 + 2, s0v)
            pltpu.async_copy(xn_c.at[s0v], r0, g0)
            _unp_dst(b0 + 1, d1v)
            pltpu.make_async_copy(xn_c.at[s1v], r1, g1).wait()
            pltpu.sync_copy(r1, acc.at[d1v], add=True)
            return carry
        lax.fori_loop(0, (bpw - 1) // 2, ebody, 0)
        _unp_dst(bpw - 1, d0v)
        pltpu.make_async_copy(xn_c.at[s0v], r0, g0).wait()
        pltpu.sync_copy(r0, acc.at[d0v], add=True)

        plsc.subcore_barrier()
        for i in range(nchk):
            ro = pl.multiple_of(sid * rps + i * _WR, 16)
            pltpu.sync_copy(acc.at[pl.ds(ro, _WR), :], r0.at[pl.ds(0, _WR), :])
            pltpu.sync_copy(r0.at[pl.ds(0, _WR), :],
                            out.at[cid, c, pl.ds(ro, _WR), :])


def _spmm_call(tab, pk3, n_pad, bpw):
    n_chunks = tab.shape[0]
    body = functools.partial(_spmm_body, n_pad, bpw, n_chunks)
    return pl.kernel(
        body,
        out_type=jax.ShapeDtypeStruct((_NC, n_chunks, n_pad, 128), _F32),
        mesh=_mesh(),
        scratch_types=(
            [pltpu.VMEM((bpw, _B), jnp.int32)]
            + [pltpu.VMEM((_B,), jnp.int32) for _ in range(6)]
            + [pltpu.VMEM((_B, 128), _F32) for _ in range(3)]
            + [pltpu.VMEM_SHARED((n_pad, 128), _F32)]
            + [pltpu.SemaphoreType.DMA for _ in range(6)]
        ),
    )(tab, pk3)


def _norm_body(deg4_ref, x_ref, xn_ref, io_ref, ii_ref):
    d = deg4_ref[...]
    dout = d[:, 0:1] + d[:, 2:3]
    din = d[:, 1:2] + d[:, 3:4]
    io = lax.rsqrt(jnp.maximum(dout, 1.0))
    ii = lax.rsqrt(jnp.maximum(din, 1.0))
    xn_ref[...] = x_ref[...] * io
    io_ref[...] = io
    ii_ref[...] = ii


def _norm_call(deg4, x, n, rb):
    grid = (n // rb,)
    return pl.pallas_call(
        _norm_body,
        grid=grid,
        in_specs=[
            pl.BlockSpec((rb, 4), lambda i: (i, 0)),
            pl.BlockSpec((rb, 128), lambda i: (i, 0)),
        ],
        out_specs=[
            pl.BlockSpec((rb, 128), lambda i: (i, 0)),
            pl.BlockSpec((rb, 1), lambda i: (i, 0)),
            pl.BlockSpec((rb, 1), lambda i: (i, 0)),
        ],
        out_shape=[
            jax.ShapeDtypeStruct((n, 128), _F32),
            jax.ShapeDtypeStruct((n, 1), _F32),
            jax.ShapeDtypeStruct((n, 1), _F32),
        ],
    )(deg4, x)


def _l1_body(u2_ref, w1_ref, b1_ref, ii_ref, io_ref, zn_ref):
    um = u2_ref[0] + u2_ref[1]
    y = jnp.dot(um, w1_ref[...], preferred_element_type=_F32,
                precision=lax.Precision.HIGHEST)
    y = y * ii_ref[...] + b1_ref[...]
    zn_ref[0] = jnp.maximum(y, 0.0) * io_ref[...]


def _l1_call(u2, w1, b1r, ii, io, n, rb):
    grid = (4, n // rb)
    return pl.pallas_call(
        _l1_body,
        grid=grid,
        in_specs=[
            pl.BlockSpec((2, rb, 128), lambda k, i: (0, i, 0)),
            pl.BlockSpec((128, 128), lambda k, i: (0, k)),
            pl.BlockSpec((1, 128), lambda k, i: (0, k)),
            pl.BlockSpec((rb, 1), lambda k, i: (i, 0)),
            pl.BlockSpec((rb, 1), lambda k, i: (i, 0)),
        ],
        out_specs=pl.BlockSpec((1, rb, 128), lambda k, i: (k, i, 0)),
        out_shape=jax.ShapeDtypeStruct((4, n, 128), _F32),
    )(u2, w1, b1r, ii, io)


def _l2_body(v2_ref, w2_ref, b2_ref, ii_ref, out_ref):
    acc = jnp.zeros(out_ref.shape, out_ref.dtype)
    for k in range(4):
        vk = v2_ref[0, k] + v2_ref[1, k]
        acc = acc + jnp.dot(vk, w2_ref[k], preferred_element_type=_F32,
                            precision=lax.Precision.HIGHEST)
    out_ref[...] = acc * ii_ref[...] + b2_ref[...]


def _l2_call(v2, w2r, b2r, ii, n, rb):
    grid = (n // rb,)
    return pl.pallas_call(
        _l2_body,
        grid=grid,
        in_specs=[
            pl.BlockSpec((2, 4, rb, 128), lambda i: (0, 0, i, 0)),
            pl.BlockSpec((4, 128, 512), lambda i: (0, 0, 0)),
            pl.BlockSpec((1, 512), lambda i: (0, 0)),
            pl.BlockSpec((rb, 1), lambda i: (i, 0)),
        ],
        out_specs=pl.BlockSpec((rb, 512), lambda i: (i, 0)),
        out_shape=jax.ShapeDtypeStruct((n, 512), _F32),
    )(v2, w2r, b2r, ii)


def kernel(features, edge_index, W1, b1, W2, b2):
    n, d_in = features.shape
    e = edge_index.shape[1]
    d_h = W1.shape[1]
    assert d_in == 128 and d_h == 512
    assert e % (_NW * _B) == 0
    bpw = e // (_NW * _B)
    n_pad = ((n + _NS * 128 - 1) // (_NS * 128)) * (_NS * 128)
    rb = 400
    assert n % rb == 0

    src = edge_index[0]
    dst = edge_index[1]
    srcs3 = src.reshape(_NW, bpw, _B)
    dsts3 = dst.reshape(_NW, bpw, _B)
    pk3 = (src + dst * 16384).reshape(_NW, bpw, _B)

    deg = _deg_call(srcs3, dsts3, n_pad, bpw)
    deg4 = jnp.transpose(deg.reshape(4, n_pad)[:, :n])

    xn, io, ii = _norm_call(deg4, features, n, rb)

    u2 = _spmm_call(xn.reshape(1, n, 128), pk3, n_pad, bpw)
    u2 = u2.reshape(_NC, n_pad, 128)

    zn = _l1_call(u2, W1, b1.reshape(1, d_h), ii, io, n, rb)

    v2 = _spmm_call(zn, pk3, n_pad, bpw)

    out = _l2_call(v2, W2.reshape(4, 128, d_h), b2.reshape(1, d_h), ii, n, rb)
    return out
